# baseline (speedup 1.0000x reference)
import functools

import jax
import jax.numpy as jnp
from jax import lax
from jax.experimental import pallas as pl
from jax.experimental.pallas import tpu as pltpu
from jax.experimental.pallas import tpu_sc as plsc

NC = 2
NS = 16
NW = NC * NS
LANES = 16
CHUNK = 128
DEG_W = 16
PAD = 8


def _vmesh():
    return plsc.VectorSubcoreMesh(core_axis_name="c", subcore_axis_name="s")


ZROWS = 32


def _fill_zero(ref, rows, width):
    zero = jnp.zeros((LANES,), jnp.float32)

    @pl.loop(0, rows)
    def _(i):
        @pl.loop(0, width, step=LANES)
        def _(j):
            ref[i, pl.ds(j, LANES)] = zero


def _zero_acc(acc_sh, zero_v, n1, sid):
    nch = n1 // ZROWS
    tail = n1 % ZROWS

    @pl.loop(sid, nch, step=NS)
    def _(c):
        pltpu.sync_copy(zero_v, acc_sh.at[pl.ds(c * ZROWS, ZROWS)])

    if tail:
        @pl.when(sid == 0)
        def _():
            pltpu.sync_copy(zero_v.at[pl.ds(0, tail)],
                            acc_sh.at[pl.ds(nch * ZROWS, tail)])


def _copy_out(acc_sh, out_hbm, n1, cid, sid):
    nch = n1 // CHUNK
    tail = n1 % CHUNK
    base = cid * n1

    @pl.loop(sid, nch, step=NS)
    def _(c):
        pltpu.sync_copy(acc_sh.at[pl.ds(c * CHUNK, CHUNK)],
                        out_hbm.at[pl.ds(base + c * CHUNK, CHUNK)])

    if tail:
        @pl.when(sid == 0)
        def _():
            pltpu.sync_copy(acc_sh.at[pl.ds(nch * CHUNK, tail)],
                            out_hbm.at[pl.ds(base + nch * CHUNK, tail)])


def _sc_degree(dst2d, n1):
    nch_w = dst2d.shape[0] // NW

    @functools.partial(
        pl.kernel,
        out_type=jax.ShapeDtypeStruct((NC * n1, DEG_W), jnp.float32),
        mesh=_vmesh(),
        scratch_types=[
            pltpu.VMEM((nch_w, CHUNK), jnp.int32),
            pltpu.VMEM((CHUNK, DEG_W), jnp.float32),
            pltpu.VMEM((ZROWS, DEG_W), jnp.float32),
            pltpu.VMEM_SHARED((n1, DEG_W), jnp.float32),
        ],
    )
    def deg_kernel(dst_hbm, out_hbm, didx_v, ones_v, zero_v, acc_sh):
        cid = lax.axis_index("c")
        sid = lax.axis_index("s")
        wid = sid * NC + cid
        wstart = wid * nch_w

        one = jnp.full((LANES,), 1.0, jnp.float32)

        @pl.loop(0, CHUNK)
        def _(i):
            ones_v[i, :] = one

        pltpu.sync_copy(dst_hbm.at[pl.ds(wstart, nch_w)], didx_v)

        _fill_zero(zero_v, ZROWS, DEG_W)
        _zero_acc(acc_sh, zero_v, n1, sid)
        plsc.subcore_barrier()

        @pl.loop(0, nch_w)
        def _(j):
            pltpu.sync_copy(ones_v, acc_sh.at[didx_v.at[j]], add=True)

        plsc.subcore_barrier()
        _copy_out(acc_sh, out_hbm, n1, cid, sid)

    return deg_kernel(dst2d)


def _sc_edge_scatter(g, src2d, dst2d):
    n1, d = g.shape
    nch_w = src2d.shape[0] // NW
    nh = nch_w // 2
    assert nh >= 2 and nh % 2 == 0 and nh % 8 == 0

    @functools.partial(
        pl.kernel,
        out_type=jax.ShapeDtypeStruct((NC * n1, d), jnp.float32),
        mesh=_vmesh(),
        compiler_params=pltpu.CompilerParams(use_tc_tiling_on_sc=False),
        scratch_types=[
            pltpu.VMEM((nh, CHUNK), jnp.int32),
            pltpu.VMEM((nh, CHUNK), jnp.int32),
            pltpu.VMEM((CHUNK,), jnp.int32),
            pltpu.VMEM((CHUNK,), jnp.int32),
            pltpu.VMEM((CHUNK, d), jnp.float32),
            pltpu.VMEM((CHUNK, d), jnp.float32),
            pltpu.VMEM((ZROWS, d), jnp.float32),
            pltpu.VMEM_SHARED((n1, d), jnp.float32),
            pltpu.SemaphoreType.DMA,
            pltpu.SemaphoreType.DMA,
        ],
    )
    def edge_kernel(g_hbm, src_hbm, dst_hbm, out_hbm,
                    sidx_v, didx_v, ddx0, ddx1, rows0, rows1, zero_v, acc_sh,
                    sem0, sem1):
        cid = lax.axis_index("c")
        sid = lax.axis_index("s")
        wid = sid * NC + cid
        wstart = wid * nch_w

        _fill_zero(zero_v, ZROWS, d)
        _zero_acc(acc_sh, zero_v, n1, sid)

        def gather(j, rows, sem):
            return pltpu.async_copy(g_hbm.at[sidx_v.at[j]], rows, sem)

        def stage_didx(j, ddx):
            for k in range(CHUNK // LANES):
                ddx[pl.ds(k * LANES, LANES)] = didx_v[j, pl.ds(k * LANES, LANES)]

        def scatter(ddx, rows):
            pltpu.sync_copy(rows, acc_sh.at[ddx], add=True)

        for h in range(2):
            pltpu.sync_copy(src_hbm.at[pl.ds(wstart + h * nh, nh)], sidx_v)
            pltpu.sync_copy(dst_hbm.at[pl.ds(wstart + h * nh, nh)], didx_v)
            if h == 0:
                plsc.subcore_barrier()

            gather(0, rows0, sem0).wait()

            @pl.loop(0, (nh - 2) // 2)
            def _(k):
                j = 2 * k
                d1 = gather(j + 1, rows1, sem1)
                stage_didx(j, ddx0)
                scatter(ddx0, rows0)
                d1.wait()
                d2 = gather(j + 2, rows0, sem0)
                stage_didx(j + 1, ddx1)
                scatter(ddx1, rows1)
                d2.wait()

            d1 = gather(nh - 1, rows1, sem1)
            stage_didx(nh - 2, ddx0)
            scatter(ddx0, rows0)
            d1.wait()
            stage_didx(nh - 1, ddx1)
            scatter(ddx1, rows1)

        plsc.subcore_barrier()
        _copy_out(acc_sh, out_hbm, n1, cid, sid)

    return edge_kernel(g, src2d, dst2d)


def _tc_matmul(x, w):
    n = x.shape[0]
    dout = w.shape[1]

    def body(x_ref, w_ref, o_ref):
        o_ref[...] = jnp.dot(x_ref[...], w_ref[...],
                             preferred_element_type=jnp.float32)

    return pl.pallas_call(
        body,
        out_shape=jax.ShapeDtypeStruct((n, dout), jnp.float32),
    )(x, w)


def _tc_scale(h, degp, n1):
    n, d = h.shape

    def body(h_ref, degp_ref, g_ref, dinv_ref):
        deg = 1.0 + degp_ref[:n, 0] + degp_ref[n1:n1 + n, 0]
        dinv = lax.rsqrt(deg)
        dinv_ref[...] = dinv[:, None]
        g_ref[pl.ds(0, n), :] = h_ref[...] * dinv[:, None]
        g_ref[pl.ds(n, n1 - n), :] = jnp.zeros((n1 - n, d), jnp.float32)

    return pl.pallas_call(
        body,
        out_shape=(
            jax.ShapeDtypeStruct((n1, d), jnp.float32),
            jax.ShapeDtypeStruct((n, 1), jnp.float32),
        ),
    )(h, degp)


def _tc_mid(s1, g1, dinv, b1, w2):
    n1, d = g1.shape
    n = dinv.shape[0]
    dout = w2.shape[1]

    def body(s_ref, g_ref, dinv_ref, b_ref, w_ref, o_ref):
        acc = s_ref[:n, :] + s_ref[n1:n1 + n, :] + g_ref[:n, :]
        z = jnp.maximum(acc * dinv_ref[...] + b_ref[...][None, :], 0.0)
        o_ref[pl.ds(0, n), :] = jnp.dot(
            z, w_ref[...], preferred_element_type=jnp.float32) * dinv_ref[...]
        o_ref[pl.ds(n, n1 - n), :] = jnp.zeros((n1 - n, dout), jnp.float32)

    return pl.pallas_call(
        body,
        out_shape=jax.ShapeDtypeStruct((n1, dout), jnp.float32),
    )(s1, g1, dinv, b1, w2)


def _tc_final(s2, g2, dinv, b2):
    n1, d = g2.shape
    n = dinv.shape[0]

    def body(s_ref, g_ref, dinv_ref, b_ref, o_ref):
        acc = s_ref[:n, :] + s_ref[n1:n1 + n, :] + g_ref[:n, :]
        o_ref[...] = jnp.maximum(acc * dinv_ref[...] + b_ref[...][None, :], 0.0)

    return pl.pallas_call(
        body,
        out_shape=jax.ShapeDtypeStruct((n, d), jnp.float32),
    )(s2, g2, dinv, b2)


@jax.jit
def kernel(x, edge_index, W1, b1, W2, b2):
    n = x.shape[0]
    n1 = n + PAD
    e = edge_index.shape[1]
    align = CHUNK * 8 * NW
    e_pad = -(-e // align) * align
    npad = e_pad - e
    src2d = jnp.concatenate(
        [edge_index[0], jnp.full((npad,), n, jnp.int32)]
    ).reshape(e_pad // CHUNK, CHUNK)
    dst_e = jnp.concatenate(
        [edge_index[1], jnp.arange(npad, dtype=jnp.int32) % n]
    ).reshape(e_pad // CHUNK, CHUNK)
    dst_d = jnp.concatenate(
        [edge_index[1], jnp.full((npad,), n, jnp.int32)]
    ).reshape(e_pad // CHUNK, CHUNK)

    degp = _sc_degree(dst_d, n1)
    h1 = _tc_matmul(x, W1)
    g1, dinv = _tc_scale(h1, degp, n1)
    s1 = _sc_edge_scatter(g1, src2d, dst_e)
    g2 = _tc_mid(s1, g1, dinv, b1, W2)
    s2 = _sc_edge_scatter(g2, src2d, dst_e)
    return _tc_final(s2, g2, dinv, b2)


# speedup vs baseline: 1.1292x; 1.1292x over previous
import functools

import numpy as np

import jax
import jax.numpy as jnp
from jax import lax
from jax.experimental import pallas as pl
from jax.experimental.pallas import tpu as pltpu
from jax.experimental.pallas import tpu_sc as plsc

NC = 2
NS = 16
NW = NC * NS
LANES = 16
CHUNK = 128
DEG_W = 16
PAD = 8


def _vmesh():
    return plsc.VectorSubcoreMesh(core_axis_name="c", subcore_axis_name="s")


ZROWS = 32


def _fill_zero(ref, rows, width):
    zero = jnp.zeros((LANES,), jnp.float32)

    @pl.loop(0, rows)
    def _(i):
        @pl.loop(0, width, step=LANES)
        def _(j):
            ref[i, pl.ds(j, LANES)] = zero


def _zero_acc(acc_sh, zero_v, n1, sid):
    nch = n1 // ZROWS
    tail = n1 % ZROWS

    @pl.loop(sid, nch, step=NS)
    def _(c):
        pltpu.sync_copy(zero_v, acc_sh.at[pl.ds(c * ZROWS, ZROWS)])

    if tail:
        @pl.when(sid == 0)
        def _():
            pltpu.sync_copy(zero_v.at[pl.ds(0, tail)],
                            acc_sh.at[pl.ds(nch * ZROWS, tail)])


def _copy_out(acc_sh, out_hbm, n1, cid, sid):
    nch = n1 // CHUNK
    tail = n1 % CHUNK
    base = cid * n1

    @pl.loop(sid, nch, step=NS)
    def _(c):
        pltpu.sync_copy(acc_sh.at[pl.ds(c * CHUNK, CHUNK)],
                        out_hbm.at[pl.ds(base + c * CHUNK, CHUNK)])

    if tail:
        @pl.when(sid == 0)
        def _():
            pltpu.sync_copy(acc_sh.at[pl.ds(nch * CHUNK, tail)],
                            out_hbm.at[pl.ds(base + nch * CHUNK, tail)])


def _sc_degree(dst2d, n1):
    nch_w = dst2d.shape[0] // NW

    @functools.partial(
        pl.kernel,
        out_type=jax.ShapeDtypeStruct((NC * n1, DEG_W), jnp.float32),
        mesh=_vmesh(),
        scratch_types=[
            pltpu.VMEM((nch_w, CHUNK), jnp.int32),
            pltpu.VMEM((CHUNK, DEG_W), jnp.float32),
            pltpu.VMEM((ZROWS, DEG_W), jnp.float32),
            pltpu.VMEM_SHARED((n1, DEG_W), jnp.float32),
        ],
    )
    def deg_kernel(dst_hbm, out_hbm, didx_v, ones_v, zero_v, acc_sh):
        cid = lax.axis_index("c")
        sid = lax.axis_index("s")
        wid = sid * NC + cid
        wstart = wid * nch_w

        one = jnp.full((LANES,), 1.0, jnp.float32)

        @pl.loop(0, CHUNK)
        def _(i):
            ones_v[i, :] = one

        pltpu.sync_copy(dst_hbm.at[pl.ds(wstart, nch_w)], didx_v)

        _fill_zero(zero_v, ZROWS, DEG_W)
        _zero_acc(acc_sh, zero_v, n1, sid)
        plsc.subcore_barrier()

        @pl.loop(0, nch_w)
        def _(j):
            pltpu.sync_copy(ones_v, acc_sh.at[didx_v.at[j]], add=True)

        plsc.subcore_barrier()
        _copy_out(acc_sh, out_hbm, n1, cid, sid)

    return deg_kernel(dst2d)


def _sc_edge_scatter(g, src2d, dst2d):
    n1, d = g.shape
    nch_w = src2d.shape[0] // NW
    nh = nch_w // 2
    assert nh >= 2 and nh % 2 == 0 and nh % 8 == 0

    @functools.partial(
        pl.kernel,
        out_type=jax.ShapeDtypeStruct((NC * n1, d), jnp.float32),
        mesh=_vmesh(),
        compiler_params=pltpu.CompilerParams(use_tc_tiling_on_sc=False),
        scratch_types=[
            pltpu.VMEM((nh, CHUNK), jnp.int32),
            pltpu.VMEM((nh, CHUNK), jnp.int32),
            pltpu.VMEM((CHUNK,), jnp.int32),
            pltpu.VMEM((CHUNK,), jnp.int32),
            pltpu.VMEM((CHUNK, d), jnp.float32),
            pltpu.VMEM((CHUNK, d), jnp.float32),
            pltpu.VMEM((ZROWS, d), jnp.float32),
            pltpu.VMEM_SHARED((n1, d), jnp.float32),
            pltpu.SemaphoreType.DMA,
            pltpu.SemaphoreType.DMA,
        ],
    )
    def edge_kernel(g_hbm, src_hbm, dst_hbm, out_hbm,
                    sidx_v, didx_v, ddx0, ddx1, rows0, rows1, zero_v, acc_sh,
                    sem0, sem1):
        cid = lax.axis_index("c")
        sid = lax.axis_index("s")
        wid = sid * NC + cid
        wstart = wid * nch_w

        _fill_zero(zero_v, ZROWS, d)
        _zero_acc(acc_sh, zero_v, n1, sid)

        def start_gather(j, rows, sem):
            pltpu.async_copy(g_hbm.at[sidx_v.at[j]], rows, sem)

        def wait_gather(j, rows, sem):
            pltpu.make_async_copy(g_hbm.at[sidx_v.at[j]], rows, sem).wait()

        def stage_didx(j, ddx):
            for k in range(CHUNK // LANES):
                ddx[pl.ds(k * LANES, LANES)] = didx_v[j, pl.ds(k * LANES, LANES)]

        def scatter(ddx, rows):
            pltpu.sync_copy(rows, acc_sh.at[ddx], add=True)

        for h in range(2):
            pltpu.sync_copy(src_hbm.at[pl.ds(wstart + h * nh, nh)], sidx_v)
            pltpu.sync_copy(dst_hbm.at[pl.ds(wstart + h * nh, nh)], didx_v)
            if h == 0:
                plsc.subcore_barrier()

            start_gather(0, rows0, sem0)

            @pl.loop(0, (nh - 2) // 2)
            def _(k):
                j = 2 * k
                start_gather(j + 1, rows1, sem1)
                stage_didx(j, ddx0)
                wait_gather(j, rows0, sem0)
                scatter(ddx0, rows0)
                start_gather(j + 2, rows0, sem0)
                stage_didx(j + 1, ddx1)
                wait_gather(j + 1, rows1, sem1)
                scatter(ddx1, rows1)

            start_gather(nh - 1, rows1, sem1)
            stage_didx(nh - 2, ddx0)
            wait_gather(nh - 2, rows0, sem0)
            scatter(ddx0, rows0)
            stage_didx(nh - 1, ddx1)
            wait_gather(nh - 1, rows1, sem1)
            scatter(ddx1, rows1)

        plsc.subcore_barrier()
        _copy_out(acc_sh, out_hbm, n1, cid, sid)

    return edge_kernel(g, src2d, dst2d)


def _tc_matmul(x, w):
    n = x.shape[0]
    dout = w.shape[1]

    def body(x_ref, w_ref, o_ref):
        o_ref[...] = jnp.dot(x_ref[...], w_ref[...],
                             preferred_element_type=jnp.float32)

    return pl.pallas_call(
        body,
        out_shape=jax.ShapeDtypeStruct((n, dout), jnp.float32),
    )(x, w)


def _tc_scale(h, degp, n1):
    n, d = h.shape

    def body(h_ref, degp_ref, g_ref, dinv_ref):
        deg = 1.0 + degp_ref[:n, 0] + degp_ref[n1:n1 + n, 0]
        dinv = lax.rsqrt(deg)
        dinv_ref[...] = dinv[:, None]
        g_ref[pl.ds(0, n), :] = h_ref[...] * dinv[:, None]
        g_ref[pl.ds(n, n1 - n), :] = jnp.zeros((n1 - n, d), jnp.float32)

    return pl.pallas_call(
        body,
        out_shape=(
            jax.ShapeDtypeStruct((n1, d), jnp.float32),
            jax.ShapeDtypeStruct((n, 1), jnp.float32),
        ),
    )(h, degp)


def _tc_mid(s1, g1, dinv, b1, w2):
    n1, d = g1.shape
    n = dinv.shape[0]
    dout = w2.shape[1]

    def body(s_ref, g_ref, dinv_ref, b_ref, w_ref, o_ref):
        acc = s_ref[:n, :] + s_ref[n1:n1 + n, :] + g_ref[:n, :]
        z = jnp.maximum(acc * dinv_ref[...] + b_ref[...][None, :], 0.0)
        o_ref[pl.ds(0, n), :] = jnp.dot(
            z, w_ref[...], preferred_element_type=jnp.float32) * dinv_ref[...]
        o_ref[pl.ds(n, n1 - n), :] = jnp.zeros((n1 - n, dout), jnp.float32)

    return pl.pallas_call(
        body,
        out_shape=jax.ShapeDtypeStruct((n1, dout), jnp.float32),
    )(s1, g1, dinv, b1, w2)


def _tc_final(s2, g2, dinv, b2):
    n1, d = g2.shape
    n = dinv.shape[0]

    def body(s_ref, g_ref, dinv_ref, b_ref, o_ref):
        acc = s_ref[:n, :] + s_ref[n1:n1 + n, :] + g_ref[:n, :]
        o_ref[...] = jnp.maximum(acc * dinv_ref[...] + b_ref[...][None, :], 0.0)

    return pl.pallas_call(
        body,
        out_shape=jax.ShapeDtypeStruct((n, d), jnp.float32),
    )(s2, g2, dinv, b2)


@jax.jit
def kernel(x, edge_index, W1, b1, W2, b2):
    n = x.shape[0]
    n1 = n + PAD
    e = edge_index.shape[1]
    align = CHUNK * 8 * NW
    e_pad = -(-e // align) * align
    npad = e_pad - e
    src2d = jnp.concatenate(
        [edge_index[0], jnp.full((npad,), n, jnp.int32)]
    ).reshape(e_pad // CHUNK, CHUNK)
    dst_e = jnp.concatenate(
        [edge_index[1], jnp.arange(npad, dtype=jnp.int32) % n]
    ).reshape(e_pad // CHUNK, CHUNK)
    dst_d = jnp.concatenate(
        [edge_index[1], jnp.full((npad,), n, jnp.int32)]
    ).reshape(e_pad // CHUNK, CHUNK)

    nch = e_pad // CHUNK
    nreal = e // CHUNK
    per_w = nch // NW
    base, extra = divmod(nreal, NW)
    perm = []
    r0, dmy = 0, nreal
    for w in range(NW):
        cnt = base + (1 if w < extra else 0)
        perm.extend(range(r0, r0 + cnt))
        r0 += cnt
        perm.extend(range(dmy, dmy + per_w - cnt))
        dmy += per_w - cnt
    perm = np.asarray(perm, np.int32)
    src2d, dst_e, dst_d = src2d[perm], dst_e[perm], dst_d[perm]

    degp = _sc_degree(dst_d, n1)
    h1 = _tc_matmul(x, W1)
    g1, dinv = _tc_scale(h1, degp, n1)
    s1 = _sc_edge_scatter(g1, src2d, dst_e)
    g2 = _tc_mid(s1, g1, dinv, b1, W2)
    s2 = _sc_edge_scatter(g2, src2d, dst_e)
    return _tc_final(s2, g2, dinv, b2)
